# baseline (speedup 1.0000x reference)
import functools

import jax
import jax.numpy as jnp
from jax import lax
from jax.experimental import pallas as pl
from jax.experimental.pallas import tpu as pltpu
from jax.experimental.pallas import tpu_sc as plsc

N = 100000
E = 1600000
IN_C = 24
HID = 256
OUT_C = 2
NUM_GRAPHS = 64

NC = 2
NS = 16
L = 16

EPAD = 1638400
NP = 101120
NPAD_ROWS = 1024
RPT_Z = NP // NS
RPT_W = N // NS
ZCHUNK = 395
W = 1024


_VIEW_LEN = 8 * (N - 1) + 1


def _propagate_body(tab, out, k, ebase, nw, s, sidx, didx, rows, zbuf,
                    acc, src_hbm, dst_hbm):
    z0 = s * RPT_Z
    for zz in range(RPT_Z // ZCHUNK):
        pltpu.sync_copy(zbuf, acc.at[pl.ds(z0 + zz * ZCHUNK, ZCHUNK)])
    plsc.subcore_barrier()

    gview = tab.at[pl.ds(k, _VIEW_LEN)]

    def window(w, _):
        base = ebase + w * W
        pltpu.sync_copy(src_hbm.at[pl.ds(base, W)], sidx)
        pltpu.sync_copy(dst_hbm.at[pl.ds(base, W)], didx)
        pltpu.sync_copy(gview.at[sidx], rows)
        pltpu.sync_copy(rows, acc.at[didx], add=True)
        return ()
    lax.fori_loop(0, nw, window, ())
    plsc.subcore_barrier()

    r0 = s * RPT_W
    pltpu.sync_copy(acc.at[pl.ds(r0, RPT_W)],
                    out.at[pl.ds(r0, RPT_W), pl.ds(k, 1)])


_SC_SCRATCH = [
    pltpu.VMEM((W,), jnp.int32),
    pltpu.VMEM((W,), jnp.int32),
    pltpu.VMEM((W, 1, L), jnp.float32),
    pltpu.VMEM((ZCHUNK, 1, L), jnp.float32),
    pltpu.VMEM_SHARED((NP, 1, L), jnp.float32),
]

_MESH = plsc.VectorSubcoreMesh(
    core_axis_name="c", subcore_axis_name="s", num_cores=NC, num_subcores=NS)
_SC_PARAMS = pltpu.CompilerParams(use_tc_tiling_on_sc=False)


def _fill_zeros(zbuf):
    def fz(i, _):
        zbuf[i, 0, :] = jnp.zeros((L,), jnp.float32)
        return ()
    lax.fori_loop(0, ZCHUNK, fz, ())


@functools.partial(
    pl.kernel,
    out_type=(jax.ShapeDtypeStruct((N, 8, L), jnp.float32),
              jax.ShapeDtypeStruct((N, 8, L), jnp.float32)),
    mesh=_MESH, compiler_params=_SC_PARAMS, scratch_types=_SC_SCRATCH)
def _prop256(tab_l, tab_r, src_hbm, dst_hbm, out_l, out_r,
             sidx, didx, rows, zbuf, acc):
    c = lax.axis_index("c")
    s = lax.axis_index("s")
    _fill_zeros(zbuf)
    ebase = s * (EPAD // NS)
    nw = (EPAD // NS) // W

    @pl.when(c == 0)
    def _():
        for k in range(8):
            _propagate_body(tab_l, out_l, k, ebase, nw, s, sidx, didx,
                            rows, zbuf, acc, src_hbm, dst_hbm)

    @pl.when(c == 1)
    def _():
        for k in range(8):
            _propagate_body(tab_r, out_r, k, ebase, nw, s, sidx, didx,
                            rows, zbuf, acc, src_hbm, dst_hbm)


@functools.partial(
    pl.kernel,
    out_type=jax.ShapeDtypeStruct((N, 8, L), jnp.float32),
    mesh=_MESH, compiler_params=_SC_PARAMS, scratch_types=_SC_SCRATCH)
def _prop32(tab, src_hbm, dst_hbm, out, sidx, didx, rows, zbuf, acc):
    c = lax.axis_index("c")
    s = lax.axis_index("s")
    _fill_zeros(zbuf)
    ebase = s * (EPAD // NS)
    nw = (EPAD // NS) // W

    @pl.when(c == 0)
    def _():
        _propagate_body(tab, out, 0, ebase, nw, s, sidx, didx, rows, zbuf,
                        acc, src_hbm, dst_hbm)

    @pl.when(c == 1)
    def _():
        _propagate_body(tab, out, 1, ebase, nw, s, sidx, didx, rows, zbuf,
                        acc, src_hbm, dst_hbm)


@functools.partial(
    pl.kernel,
    out_type=(jax.ShapeDtypeStruct((N, 8, L), jnp.float32),
              jax.ShapeDtypeStruct((N, 8, L), jnp.float32)),
    mesh=_MESH, compiler_params=_SC_PARAMS, scratch_types=_SC_SCRATCH)
def _prop16(tab, src_hbm, dst_hbm, out_a, out_b, sidx, didx, rows, zbuf, acc):
    c = lax.axis_index("c")
    s = lax.axis_index("s")
    _fill_zeros(zbuf)
    ept = EPAD // (NC * NS)
    nw = ept // W

    @pl.when(c == 0)
    def _():
        _propagate_body(tab, out_a, 0, s * ept, nw, s, sidx, didx, rows,
                        zbuf, acc, src_hbm, dst_hbm)

    @pl.when(c == 1)
    def _():
        _propagate_body(tab, out_b, 0, EPAD // NC + s * ept, nw, s, sidx,
                        didx, rows, zbuf, acc, src_hbm, dst_hbm)


_B = 1024
_GRID = (N + _B - 1) // _B
_F = _B * 128


def _row_spec():
    return pl.BlockSpec((_B * 128,), lambda i: (i,))


def _col_spec():
    return pl.BlockSpec((_B, 1), lambda i: (i, 0))


def _full(shape):
    return pl.BlockSpec(shape, lambda i: (0, 0))


def _tc_prep(da1, db1, x128):
    def body(da_ref, db_ref, x_ref, dis_ref, g0_ref):
        da = da_ref[...].reshape(_B, 128)
        db = db_ref[...].reshape(_B, 128)
        deg = 1.0 + da[:, 0:1] + db[:, 0:1]
        dis = lax.rsqrt(deg)
        dis_ref[...] = dis
        g0_ref[...] = (dis * x_ref[...]).reshape(_F)

    return pl.pallas_call(
        body,
        grid=(_GRID,),
        in_specs=[_row_spec(), _row_spec(),
                  pl.BlockSpec((_B, 128), lambda i: (i, 0))],
        out_specs=[_col_spec(), _row_spec()],
        out_shape=[jax.ShapeDtypeStruct((N, 1), jnp.float32),
                   jax.ShapeDtypeStruct((N * 128,), jnp.float32)],
    )(da1, db1, x128)


def _tc_layer1(z1, g1, dis, W1pp, b1):
    def body(z_ref, g_ref, dis_ref, w_ref, b_ref, gl_ref, gr_ref):
        u = (z_ref[...] + g_ref[...]).reshape(_B, 128)
        lane = lax.broadcasted_iota(jnp.int32, (_B, 128), 1)
        u = dis_ref[...] * jnp.where(lane < 2 * L, u, 0.0)
        h = jnp.dot(u, w_ref[...], preferred_element_type=jnp.float32)
        h = dis_ref[...] * jnp.maximum(h + b_ref[...], 0.0)
        gl_ref[...] = h[:, :128].reshape(_F)
        gr_ref[...] = h[:, 128:].reshape(_F)

    return pl.pallas_call(
        body,
        grid=(_GRID,),
        in_specs=[_row_spec(), _row_spec(), _col_spec(),
                  _full((128, HID)), _full((1, HID))],
        out_specs=[_row_spec(), _row_spec()],
        out_shape=[jax.ShapeDtypeStruct((N * 128,), jnp.float32)] * 2,
    )(z1, g1, dis, W1pp, b1)


def _tc_layer(zl, zr, gl, gr, dis, Wm, b):
    def body(zl_ref, zr_ref, gl_ref, gr_ref, dis_ref, w_ref, b_ref,
             ol_ref, or_ref):
        dis = dis_ref[...]
        ul = dis * (zl_ref[...] + gl_ref[...]).reshape(_B, 128)
        ur = dis * (zr_ref[...] + gr_ref[...]).reshape(_B, 128)
        w = w_ref[...]
        h = (jnp.dot(ul, w[:128, :], preferred_element_type=jnp.float32)
             + jnp.dot(ur, w[128:, :], preferred_element_type=jnp.float32))
        h = dis * jnp.maximum(h + b_ref[...], 0.0)
        ol_ref[...] = h[:, :128].reshape(_F)
        or_ref[...] = h[:, 128:].reshape(_F)

    return pl.pallas_call(
        body,
        grid=(_GRID,),
        in_specs=[_row_spec(), _row_spec(), _row_spec(), _row_spec(),
                  _col_spec(), _full((HID, HID)), _full((1, HID))],
        out_specs=[_row_spec(), _row_spec()],
        out_shape=[jax.ShapeDtypeStruct((N * 128,), jnp.float32)] * 2,
    )(zl, zr, gl, gr, dis, Wm, b)


def _tc_layer5(zl, zr, gl, gr, dis, W5m, b5, W6m, linW128):
    def body(zl_ref, zr_ref, gl_ref, gr_ref, dis_ref, w5_ref, b5_ref,
             w6_ref, lw_ref, q_ref):
        dis = dis_ref[...]
        ul = dis * (zl_ref[...] + gl_ref[...]).reshape(_B, 128)
        ur = dis * (zr_ref[...] + gr_ref[...]).reshape(_B, 128)
        w5 = w5_ref[...]
        h = (jnp.dot(ul, w5[:128, :], preferred_element_type=jnp.float32)
             + jnp.dot(ur, w5[128:, :], preferred_element_type=jnp.float32))
        h = jnp.maximum(h + b5_ref[...], 0.0)
        wf = jnp.dot(w6_ref[...], lw_ref[...],
                     preferred_element_type=jnp.float32)
        q = dis * jnp.dot(h, wf, preferred_element_type=jnp.float32)
        q_ref[...] = q.reshape(_F)

    return pl.pallas_call(
        body,
        grid=(_GRID,),
        in_specs=[_row_spec(), _row_spec(), _row_spec(), _row_spec(),
                  _col_spec(), _full((HID, HID)), _full((1, HID)),
                  _full((HID, HID)), _full((HID, 128))],
        out_specs=_row_spec(),
        out_shape=jax.ShapeDtypeStruct((N * 128,), jnp.float32),
    )(zl, zr, gl, gr, dis, W5m, b5, W6m, linW128)


def _tc_pool(q1, qa1, qb1, dis, batch2d, b6, linW128, linb128):
    def body(q_ref, qa_ref, qb_ref, dis_ref, batch_ref, b6_ref, lw_ref,
             lb_ref, out_ref, acc_ref):
        i = pl.program_id(0)

        @pl.when(i == 0)
        def _():
            acc_ref[...] = jnp.zeros_like(acc_ref)

        v = (q_ref[...] + qa_ref[...] + qb_ref[...]).reshape(_B, 128)
        lane = lax.broadcasted_iota(jnp.int32, (_B, 128), 1)
        v = dis_ref[...] * jnp.where(lane < L, v, 0.0)
        vv = jnp.where(lane == 2, 1.0, v)
        gid = lax.broadcasted_iota(jnp.int32, (_B, NUM_GRAPHS), 1)
        rowid = i * _B + lax.broadcasted_iota(jnp.int32, (_B, 1), 0)
        oneh = ((batch_ref[...] == gid) & (rowid < N)).astype(jnp.float32)
        part = lax.dot_general(
            oneh, vv, dimension_numbers=(((0,), (0,)), ((), ())),
            preferred_element_type=jnp.float32)
        acc_ref[...] += part

        @pl.when(i == _GRID - 1)
        def _():
            acc = acc_ref[...]
            counts = jnp.maximum(acc[:, 2:3], 1.0)
            bf = jnp.dot(b6_ref[...], lw_ref[...],
                         preferred_element_type=jnp.float32) + lb_ref[...]
            logits = acc / counts + bf
            l2 = logits[:, 0:2]
            m = jnp.max(l2, axis=1, keepdims=True)
            lse = m + jnp.log(jnp.sum(jnp.exp(l2 - m), axis=1, keepdims=True))
            out_ref[...] = l2 - lse

    return pl.pallas_call(
        body,
        grid=(_GRID,),
        in_specs=[_row_spec(), _row_spec(), _row_spec(),
                  _col_spec(), _col_spec(),
                  _full((1, HID)), _full((HID, 128)), _full((1, 128))],
        out_specs=pl.BlockSpec((NUM_GRAPHS, OUT_C), lambda i: (0, 0)),
        out_shape=jax.ShapeDtypeStruct((NUM_GRAPHS, OUT_C), jnp.float32),
        scratch_shapes=[pltpu.VMEM((NUM_GRAPHS, 128), jnp.float32)],
    )(q1, qa1, qb1, dis, batch2d, b6, linW128, linb128)


def kernel(x, edge_index, batch, W1, b1, W2, b2, W3, b3, W4, b4, W5, b5,
           W6, b6, lin_W, lin_b):
    src = edge_index[0]
    dst = edge_index[1]
    pad = jnp.arange(EPAD - E, dtype=jnp.int32) & (NPAD_ROWS - 1)
    src_p = jnp.concatenate([src, pad]) * 8
    dst_p = jnp.concatenate([dst, N + pad])

    x128 = jnp.pad(x, ((0, 0), (0, 128 - IN_C)))
    W1pp = jnp.pad(W1, ((0, 128 - IN_C), (0, 0)))
    linW128 = jnp.pad(lin_W, ((0, 0), (0, 128 - OUT_C)))
    linb128 = jnp.pad(lin_b, (0, 128 - OUT_C)).reshape(1, 128)
    ones128 = jnp.ones((N, 128), jnp.float32)

    da, db = _prop16(ones128.reshape(N * 8, 1, L), src_p, dst_p)
    dis, g0 = _tc_prep(da.reshape(-1), db.reshape(-1), x128)

    z0 = _prop32(g0.reshape(N * 8, 1, L), src_p, dst_p)
    gl, gr = _tc_layer1(z0.reshape(-1), g0, dis, W1pp, b1.reshape(1, HID))

    for (Wm, b) in ((W2, b2), (W3, b3), (W4, b4)):
        zl, zr = _prop256(gl.reshape(N * 8, 1, L), gr.reshape(N * 8, 1, L),
                          src_p, dst_p)
        gl, gr = _tc_layer(zl.reshape(-1), zr.reshape(-1), gl, gr, dis,
                           Wm, b.reshape(1, HID))

    z4l, z4r = _prop256(gl.reshape(N * 8, 1, L), gr.reshape(N * 8, 1, L),
                        src_p, dst_p)
    q = _tc_layer5(z4l.reshape(-1), z4r.reshape(-1), gl, gr, dis,
                   W5, b5.reshape(1, HID), W6, linW128)

    qa, qb = _prop16(q.reshape(N * 8, 1, L), src_p, dst_p)
    return _tc_pool(q, qa.reshape(-1), qb.reshape(-1), dis,
                    batch.reshape(N, 1), b6.reshape(1, HID), linW128,
                    linb128)
